# baseline (speedup 1.0000x reference)
import functools

import jax
import jax.numpy as jnp
from jax import lax
from jax.experimental import pallas as pl
from jax.experimental.pallas import tpu as pltpu
from jax.experimental.pallas import tpu_sc as plsc

HIDDEN = 128
DEPTH = 3
ATOM_FDIM = 39
BOND_FDIM = 11
N_ATOMS = 10000
TOTAL_BONDS = 320000
MAX_NB = 32

NC, NS = 2, 16
NW = NC * NS
N_PAD = 10240
APW = N_PAD // NW
GRP = 4
NG = APW // GRP
IDX_W = GRP * MAX_NB
BF_PAD = 16
ZROW = N_ATOMS
BCHUNK = TOTAL_BONDS // 4

_mesh = plsc.VectorSubcoreMesh(
    core_axis_name="c", subcore_axis_name="s", num_cores=NC, num_subcores=NS
)


def _wid():
    return lax.axis_index("s") * NC + lax.axis_index("c")


@functools.partial(
    pl.kernel,
    out_type=jax.ShapeDtypeStruct((NW, NG, IDX_W), jnp.int32),
    mesh=_mesh,
    scratch_types=[
        pltpu.VMEM((NG, IDX_W), jnp.int32),
        pltpu.VMEM((NG, IDX_W), jnp.int32),
        pltpu.VMEM((BCHUNK,), jnp.int32),
    ],
    compiler_params=pltpu.CompilerParams(needs_layout_passes=False),
)
def _sc_compose(aout_hbm, bsrc_hbm, src_hbm, aout_v, bg_v, bond_v):
    w = _wid()
    pltpu.sync_copy(aout_hbm.at[w], aout_v)

    for p in range(TOTAL_BONDS // BCHUNK):
        pltpu.sync_copy(bsrc_hbm.at[pl.ds(p * BCHUNK, BCHUNK)], bond_v)
        lo = p * BCHUNK

        def pass_body(g, carry, lo=lo):
            for j in range(IDX_W // 16):
                sl = pl.ds(j * 16, 16)
                idx = aout_v[g, sl]
                loc = jnp.clip(idx - lo, 0, BCHUNK - 1)
                got = plsc.load_gather(bond_v, [loc])
                m = (idx >= lo) & (idx < lo + BCHUNK)
                prev = bg_v[g, sl] if lo else jnp.zeros((16,), jnp.int32)
                bg_v[g, sl] = jnp.where(m, got, prev)
            return carry

        lax.fori_loop(0, NG, pass_body, 0)

    def compose(g, carry):
        for j in range(IDX_W // 16):
            sl = pl.ds(j * 16, 16)
            b = aout_v[g, sl]
            s = bg_v[g, sl]
            sent = (jnp.int32(ZROW) + ((g * (IDX_W // 16) + j) % 15) * 16
                    + lax.iota(jnp.int32, 16))
            bg_v[g, sl] = jnp.where(b == 0, sent, s)
        return carry

    lax.fori_loop(0, NG, compose, 0)
    pltpu.sync_copy(bg_v, src_hbm.at[w])


def _make_gather_sum(n_rows):
    @functools.partial(
        pl.kernel,
        out_type=jax.ShapeDtypeStruct((NW, APW, HIDDEN), jnp.float32),
        mesh=_mesh,
        scratch_types=[
            pltpu.VMEM((NG, IDX_W), jnp.int32),
            pltpu.VMEM((IDX_W, HIDDEN), jnp.float32),
            pltpu.VMEM((IDX_W, HIDDEN), jnp.float32),
            pltpu.VMEM((APW, HIDDEN), jnp.float32),
            pltpu.SemaphoreType.DMA,
            pltpu.SemaphoreType.DMA,
        ],
    )
    def gather_sum(tab_hbm, idx_hbm, out_hbm, idx_v, buf0, buf1, out_v, sem0, sem1):
        w = _wid()
        pltpu.sync_copy(idx_hbm.at[w], idx_v)

        bufs = (buf0, buf1)
        sems = (sem0, sem1)
        pltpu.async_copy(tab_hbm.at[idx_v.at[0]], buf0, sem0)
        pltpu.async_copy(tab_hbm.at[idx_v.at[1]], buf1, sem1)

        def step(g, carry):
            for b in range(2):
                grp = 2 * g + b
                pltpu.make_async_copy(tab_hbm.at[idx_v.at[0]], bufs[b], sems[b]).wait()
                for a in range(GRP):
                    base = a * MAX_NB

                    def rbody(r, accs, b=b, base=base):
                        out = []
                        for col in range(HIDDEN // 16):
                            acc = accs[col]
                            for rr in range(4):
                                acc = acc + bufs[b][base + r * 4 + rr,
                                                    pl.ds(col * 16, 16)]
                            out.append(acc)
                        return tuple(out)

                    accs = lax.fori_loop(
                        0, MAX_NB // 4, rbody,
                        tuple(jnp.zeros((16,), jnp.float32)
                              for _ in range(HIDDEN // 16)),
                    )
                    for col in range(HIDDEN // 16):
                        out_v[grp * GRP + a, pl.ds(col * 16, 16)] = accs[col]

                @pl.when(g < NG // 2 - 1)
                def _(b=b, grp=grp):
                    pltpu.async_copy(tab_hbm.at[idx_v.at[grp + 2]], bufs[b], sems[b])

            return carry

        lax.fori_loop(0, NG // 2, step, 0)
        pltpu.sync_copy(out_v, out_hbm.at[w])

    return gather_sum


_sc_gather_sum_h = _make_gather_sum(N_PAD)
_sc_gather_sum_fb = _make_gather_sum(TOTAL_BONDS)


_TC_ROWS = 1280
_TC_GRID = N_PAD // _TC_ROWS
_FB_ROWS = 6400
_FB_GRID = TOTAL_BONDS // _FB_ROWS


def _tc_fbproj_body(fb_ref, wb_ref, out_ref):
    out_ref[...] = lax.dot_general(
        fb_ref[...], wb_ref[...], (((1,), (1,)), ((), ())),
        preferred_element_type=jnp.float32)


def _tc_fbproj(fbonds, w_b):
    return pl.pallas_call(
        _tc_fbproj_body,
        grid=(_FB_GRID,),
        in_specs=[
            pl.BlockSpec((_FB_ROWS, BOND_FDIM), lambda i: (i, 0)),
            pl.BlockSpec((HIDDEN, BOND_FDIM), lambda i: (0, 0)),
        ],
        out_specs=pl.BlockSpec((_FB_ROWS, HIDDEN), lambda i: (i, 0)),
        out_shape=jax.ShapeDtypeStruct((TOTAL_BONDS, HIDDEN), jnp.float32),
    )(fbonds, w_b)


def _tc_h0_body(fat_ref, wnin_ref, h0_ref):
    h0_ref[...] = jnp.maximum(
        lax.dot_general(fat_ref[...], wnin_ref[...], (((1,), (1,)), ((), ())),
                        preferred_element_type=jnp.float32),
        0.0,
    )


def _tc_h0(fatoms_pad, w_nin):
    return pl.pallas_call(
        _tc_h0_body,
        grid=(_TC_GRID,),
        in_specs=[
            pl.BlockSpec((_TC_ROWS, ATOM_FDIM), lambda i: (i, 0)),
            pl.BlockSpec((HIDDEN, ATOM_FDIM), lambda i: (0, 0)),
        ],
        out_specs=pl.BlockSpec((_TC_ROWS, HIDDEN), lambda i: (i, 0)),
        out_shape=jax.ShapeDtypeStruct((N_PAD, HIDDEN), jnp.float32),
    )(fatoms_pad, w_nin)


def _tc_update_body(nei_ref, h0_ref, bias_ref, wh_ref, out_ref):
    i = pl.program_id(0)
    y = jnp.maximum(
        h0_ref[...] + bias_ref[...]
        + lax.dot_general(nei_ref[...], wh_ref[...], (((1,), (1,)), ((), ())),
                          preferred_element_type=jnp.float32),
        0.0,
    )
    rows = lax.broadcasted_iota(jnp.int32, (_TC_ROWS, 1), 0) + i * _TC_ROWS
    out_ref[...] = jnp.where(rows < N_ATOMS, y, 0.0)


def _tc_update(nei, h0, bias, w_h):
    return pl.pallas_call(
        _tc_update_body,
        grid=(_TC_GRID,),
        in_specs=[
            pl.BlockSpec((_TC_ROWS, HIDDEN), lambda i: (i, 0)),
            pl.BlockSpec((_TC_ROWS, HIDDEN), lambda i: (i, 0)),
            pl.BlockSpec((_TC_ROWS, HIDDEN), lambda i: (i, 0)),
            pl.BlockSpec((HIDDEN, HIDDEN), lambda i: (0, 0)),
        ],
        out_specs=pl.BlockSpec((_TC_ROWS, HIDDEN), lambda i: (i, 0)),
        out_shape=jax.ShapeDtypeStruct((N_PAD, HIDDEN), jnp.float32),
    )(nei, h0, bias, w_h)


def kernel(fatoms, fbonds, aoutgraph, bgraph, aingraph, scope, all_bonds, W_nin, W_node):
    fatoms_pad = jnp.pad(fatoms, ((0, N_PAD - N_ATOMS), (0, 0)))
    aout_pad = jnp.pad(aoutgraph, ((0, N_PAD - N_ATOMS), (0, 0)))
    aout_g = aout_pad.reshape(NW, NG, IDX_W)
    spread = (jnp.arange(N_PAD * MAX_NB, dtype=jnp.int32) % TOTAL_BONDS
              ).reshape(N_PAD, MAX_NB)
    rows = jnp.arange(N_PAD, dtype=jnp.int32)[:, None]
    aout_bias = jnp.where(rows < N_ATOMS, aout_pad, spread).reshape(NW, NG, IDX_W)
    bsrc = all_bonds[:, 1]
    w_h = W_node[:, :HIDDEN]
    w_b = W_node[:, HIDDEN:]

    src = _sc_compose(aout_g, bsrc)
    h0 = _tc_h0(fatoms_pad, W_nin)
    nei1 = _sc_gather_sum_h(h0, src)
    fbonds_b, _ = lax.optimization_barrier((fbonds, h0))
    fb_proj = _tc_fbproj(fbonds_b, w_b)
    bias = _sc_gather_sum_fb(fb_proj, aout_bias).reshape(N_PAD, HIDDEN)

    h = _tc_update(nei1.reshape(N_PAD, HIDDEN), h0, bias, w_h)
    for _ in range(DEPTH - 1):
        nei = _sc_gather_sum_h(h, src)
        h = _tc_update(nei.reshape(N_PAD, HIDDEN), h0, bias, w_h)

    return h[:N_ATOMS].T


# speedup vs baseline: 1.2264x; 1.2264x over previous
import functools

import jax
import jax.numpy as jnp
from jax import lax
from jax.experimental import pallas as pl
from jax.experimental.pallas import tpu as pltpu
from jax.experimental.pallas import tpu_sc as plsc

HIDDEN = 128
DEPTH = 3
ATOM_FDIM = 39
BOND_FDIM = 11
N_ATOMS = 10000
TOTAL_BONDS = 320000
MAX_NB = 32

NC, NS = 2, 16
NW = NC * NS
N_PAD = 10240
APW = N_PAD // NW
GRP = 4
NG = APW // GRP
IDX_W = GRP * MAX_NB
BF_PAD = 16
ZROW = N_ATOMS
BCHUNK = TOTAL_BONDS // 4

_mesh = plsc.VectorSubcoreMesh(
    core_axis_name="c", subcore_axis_name="s", num_cores=NC, num_subcores=NS
)


def _wid():
    return lax.axis_index("s") * NC + lax.axis_index("c")


@functools.partial(
    pl.kernel,
    out_type=jax.ShapeDtypeStruct((NW, NG, IDX_W), jnp.int32),
    mesh=_mesh,
    scratch_types=[
        pltpu.VMEM((NG, IDX_W), jnp.int32),
        pltpu.VMEM((NG, IDX_W), jnp.int32),
        pltpu.VMEM((BCHUNK,), jnp.int32),
    ],
    compiler_params=pltpu.CompilerParams(needs_layout_passes=False),
)
def _sc_compose(aout_hbm, bsrc_hbm, src_hbm, aout_v, bg_v, bond_v):
    w = _wid()
    pltpu.sync_copy(aout_hbm.at[w], aout_v)

    for p in range(TOTAL_BONDS // BCHUNK):
        pltpu.sync_copy(bsrc_hbm.at[pl.ds(p * BCHUNK, BCHUNK)], bond_v)
        lo = p * BCHUNK

        def pass_body(g, carry, lo=lo):
            for j in range(IDX_W // 16):
                sl = pl.ds(j * 16, 16)
                idx = aout_v[g, sl]
                loc = jnp.clip(idx - lo, 0, BCHUNK - 1)
                got = plsc.load_gather(bond_v, [loc])
                m = (idx >= lo) & (idx < lo + BCHUNK)
                prev = bg_v[g, sl] if lo else jnp.zeros((16,), jnp.int32)
                bg_v[g, sl] = jnp.where(m, got, prev)
            return carry

        lax.fori_loop(0, NG, pass_body, 0)

    def compose(g, carry):
        for j in range(IDX_W // 16):
            sl = pl.ds(j * 16, 16)
            b = aout_v[g, sl]
            s = bg_v[g, sl]
            sent = (jnp.int32(ZROW) + ((g * (IDX_W // 16) + j) % 15) * 16
                    + lax.iota(jnp.int32, 16))
            bg_v[g, sl] = jnp.where(b == 0, sent, s)
        return carry

    lax.fori_loop(0, NG, compose, 0)
    pltpu.sync_copy(bg_v, src_hbm.at[w])


def _make_gather_sum(n_rows):
    @functools.partial(
        pl.kernel,
        out_type=jax.ShapeDtypeStruct((NW, APW, HIDDEN), jnp.float32),
        mesh=_mesh,
        scratch_types=[
            pltpu.VMEM((NG, IDX_W), jnp.int32),
            pltpu.VMEM((IDX_W, HIDDEN), jnp.float32),
            pltpu.VMEM((IDX_W, HIDDEN), jnp.float32),
            pltpu.VMEM((APW, HIDDEN), jnp.float32),
            pltpu.SemaphoreType.DMA,
            pltpu.SemaphoreType.DMA,
        ],
    )
    def gather_sum(tab_hbm, idx_hbm, out_hbm, idx_v, buf0, buf1, out_v, sem0, sem1):
        w = _wid()
        pltpu.sync_copy(idx_hbm.at[w], idx_v)

        bufs = (buf0, buf1)
        sems = (sem0, sem1)
        pltpu.async_copy(tab_hbm.at[idx_v.at[0]], buf0, sem0)
        pltpu.async_copy(tab_hbm.at[idx_v.at[1]], buf1, sem1)

        def step(g, carry):
            for b in range(2):
                grp = 2 * g + b
                pltpu.make_async_copy(tab_hbm.at[idx_v.at[0]], bufs[b], sems[b]).wait()
                for a in range(GRP):
                    base = a * MAX_NB

                    def rbody(r, accs, b=b, base=base):
                        out = []
                        for col in range(HIDDEN // 16):
                            acc = accs[col]
                            for rr in range(4):
                                acc = acc + bufs[b][base + r * 4 + rr,
                                                    pl.ds(col * 16, 16)]
                            out.append(acc)
                        return tuple(out)

                    accs = lax.fori_loop(
                        0, MAX_NB // 4, rbody,
                        tuple(jnp.zeros((16,), jnp.float32)
                              for _ in range(HIDDEN // 16)),
                    )
                    for col in range(HIDDEN // 16):
                        out_v[grp * GRP + a, pl.ds(col * 16, 16)] = accs[col]

                @pl.when(g < NG // 2 - 1)
                def _(b=b, grp=grp):
                    pltpu.async_copy(tab_hbm.at[idx_v.at[grp + 2]], bufs[b], sems[b])

            return carry

        lax.fori_loop(0, NG // 2, step, 0)
        pltpu.sync_copy(out_v, out_hbm.at[w])

    return gather_sum


_sc_gather_sum_h = _make_gather_sum(N_PAD)
_sc_gather_sum_fb = _make_gather_sum(TOTAL_BONDS)


_TC_ROWS = 1280
_TC_GRID = N_PAD // _TC_ROWS
_FB_ROWS = 6400
_FB_GRID = TOTAL_BONDS // _FB_ROWS


def _tc_fbproj_body(fbt_ref, wb_ref, out_ref):
    out_ref[...] = lax.dot_general(
        fbt_ref[...], wb_ref[...], (((0,), (1,)), ((), ())),
        preferred_element_type=jnp.float32)


def _tc_fbproj(fb_t, w_b):
    return pl.pallas_call(
        _tc_fbproj_body,
        grid=(_FB_GRID,),
        in_specs=[
            pl.BlockSpec((BOND_FDIM, _FB_ROWS), lambda i: (0, i)),
            pl.BlockSpec((HIDDEN, BOND_FDIM), lambda i: (0, 0)),
        ],
        out_specs=pl.BlockSpec((_FB_ROWS, HIDDEN), lambda i: (i, 0)),
        out_shape=jax.ShapeDtypeStruct((TOTAL_BONDS, HIDDEN), jnp.float32),
    )(fb_t, w_b)


def _tc_h0_body(fat_ref, wnin_ref, h0_ref):
    h0_ref[...] = jnp.maximum(
        lax.dot_general(fat_ref[...], wnin_ref[...], (((1,), (1,)), ((), ())),
                        preferred_element_type=jnp.float32),
        0.0,
    )


def _tc_h0(fatoms_pad, w_nin):
    return pl.pallas_call(
        _tc_h0_body,
        grid=(_TC_GRID,),
        in_specs=[
            pl.BlockSpec((_TC_ROWS, ATOM_FDIM), lambda i: (i, 0)),
            pl.BlockSpec((HIDDEN, ATOM_FDIM), lambda i: (0, 0)),
        ],
        out_specs=pl.BlockSpec((_TC_ROWS, HIDDEN), lambda i: (i, 0)),
        out_shape=jax.ShapeDtypeStruct((N_PAD, HIDDEN), jnp.float32),
    )(fatoms_pad, w_nin)


def _tc_update_body(nei_ref, h0_ref, bias_ref, wh_ref, out_ref):
    i = pl.program_id(0)
    y = jnp.maximum(
        h0_ref[...] + bias_ref[...]
        + lax.dot_general(nei_ref[...], wh_ref[...], (((1,), (1,)), ((), ())),
                          preferred_element_type=jnp.float32),
        0.0,
    )
    rows = lax.broadcasted_iota(jnp.int32, (_TC_ROWS, 1), 0) + i * _TC_ROWS
    out_ref[...] = jnp.where(rows < N_ATOMS, y, 0.0)


def _tc_update(nei, h0, bias, w_h):
    return pl.pallas_call(
        _tc_update_body,
        grid=(_TC_GRID,),
        in_specs=[
            pl.BlockSpec((_TC_ROWS, HIDDEN), lambda i: (i, 0)),
            pl.BlockSpec((_TC_ROWS, HIDDEN), lambda i: (i, 0)),
            pl.BlockSpec((_TC_ROWS, HIDDEN), lambda i: (i, 0)),
            pl.BlockSpec((HIDDEN, HIDDEN), lambda i: (0, 0)),
        ],
        out_specs=pl.BlockSpec((_TC_ROWS, HIDDEN), lambda i: (i, 0)),
        out_shape=jax.ShapeDtypeStruct((N_PAD, HIDDEN), jnp.float32),
    )(nei, h0, bias, w_h)


def kernel(fatoms, fbonds, aoutgraph, bgraph, aingraph, scope, all_bonds, W_nin, W_node):
    fatoms_pad = jnp.pad(fatoms, ((0, N_PAD - N_ATOMS), (0, 0)))
    aout_pad = jnp.pad(aoutgraph, ((0, N_PAD - N_ATOMS), (0, 0)))
    aout_g = aout_pad.reshape(NW, NG, IDX_W)
    spread = (jnp.arange(N_PAD * MAX_NB, dtype=jnp.int32) % TOTAL_BONDS
              ).reshape(N_PAD, MAX_NB)
    rows = jnp.arange(N_PAD, dtype=jnp.int32)[:, None]
    aout_bias = jnp.where(rows < N_ATOMS, aout_pad, spread).reshape(NW, NG, IDX_W)
    bsrc = all_bonds[:, 1]
    w_h = W_node[:, :HIDDEN]
    w_b = W_node[:, HIDDEN:]

    src = _sc_compose(aout_g, bsrc)
    h0 = _tc_h0(fatoms_pad, W_nin)
    nei1 = _sc_gather_sum_h(h0, src)
    fb_proj = _tc_fbproj(fbonds.T, w_b)
    bias = _sc_gather_sum_fb(fb_proj, aout_bias).reshape(N_PAD, HIDDEN)

    h = _tc_update(nei1.reshape(N_PAD, HIDDEN), h0, bias, w_h)
    for _ in range(DEPTH - 1):
        nei = _sc_gather_sum_h(h, src)
        h = _tc_update(nei.reshape(N_PAD, HIDDEN), h0, bias, w_h)

    return h[:N_ATOMS].T


# speedup vs baseline: 1.2355x; 1.0074x over previous
import functools

import jax
import jax.numpy as jnp
from jax import lax
from jax.experimental import pallas as pl
from jax.experimental.pallas import tpu as pltpu
from jax.experimental.pallas import tpu_sc as plsc

HIDDEN = 128
DEPTH = 3
ATOM_FDIM = 39
BOND_FDIM = 11
N_ATOMS = 10000
TOTAL_BONDS = 320000
MAX_NB = 32

NC, NS = 2, 16
NW = NC * NS
N_PAD = 10240
APW = N_PAD // NW
GRP = 4
NG = APW // GRP
IDX_W = GRP * MAX_NB
BF_PAD = 16
ZROW = N_ATOMS
BCHUNK = TOTAL_BONDS // 4

_mesh = plsc.VectorSubcoreMesh(
    core_axis_name="c", subcore_axis_name="s", num_cores=NC, num_subcores=NS
)


def _wid():
    return lax.axis_index("s") * NC + lax.axis_index("c")


@functools.partial(
    pl.kernel,
    out_type=jax.ShapeDtypeStruct((NW, NG, IDX_W), jnp.int32),
    mesh=_mesh,
    scratch_types=[
        pltpu.VMEM((NG, IDX_W), jnp.int32),
        pltpu.VMEM((NG, IDX_W), jnp.int32),
        pltpu.VMEM((BCHUNK,), jnp.int32),
    ],
    compiler_params=pltpu.CompilerParams(needs_layout_passes=False),
)
def _sc_compose(aout_hbm, abt_hbm, src_hbm, aout_v, bg_v, bond_v):
    w = _wid()
    pltpu.sync_copy(aout_hbm.at[w], aout_v)

    for p in range(TOTAL_BONDS // BCHUNK):
        pltpu.sync_copy(abt_hbm.at[1, pl.ds(p * BCHUNK, BCHUNK)], bond_v)
        lo = p * BCHUNK

        def pass_body(g, carry, lo=lo):
            for j in range(IDX_W // 16):
                sl = pl.ds(j * 16, 16)
                idx = aout_v[g, sl]
                loc = jnp.clip(idx - lo, 0, BCHUNK - 1)
                got = plsc.load_gather(bond_v, [loc])
                m = (idx >= lo) & (idx < lo + BCHUNK)
                prev = bg_v[g, sl] if lo else jnp.zeros((16,), jnp.int32)
                bg_v[g, sl] = jnp.where(m, got, prev)
            return carry

        lax.fori_loop(0, NG, pass_body, 0)

    def compose(g, carry):
        for j in range(IDX_W // 16):
            sl = pl.ds(j * 16, 16)
            b = aout_v[g, sl]
            s = bg_v[g, sl]
            sent = (jnp.int32(ZROW) + ((g * (IDX_W // 16) + j) % 15) * 16
                    + lax.iota(jnp.int32, 16))
            bg_v[g, sl] = jnp.where(b == 0, sent, s)
        return carry

    lax.fori_loop(0, NG, compose, 0)
    pltpu.sync_copy(bg_v, src_hbm.at[w])


def _make_gather_sum(n_rows):
    @functools.partial(
        pl.kernel,
        out_type=jax.ShapeDtypeStruct((NW, APW, HIDDEN), jnp.float32),
        mesh=_mesh,
        scratch_types=[
            pltpu.VMEM((NG, IDX_W), jnp.int32),
            pltpu.VMEM((IDX_W, HIDDEN), jnp.float32),
            pltpu.VMEM((IDX_W, HIDDEN), jnp.float32),
            pltpu.VMEM((APW, HIDDEN), jnp.float32),
            pltpu.SemaphoreType.DMA,
            pltpu.SemaphoreType.DMA,
        ],
    )
    def gather_sum(tab_hbm, idx_hbm, out_hbm, idx_v, buf0, buf1, out_v, sem0, sem1):
        w = _wid()
        pltpu.sync_copy(idx_hbm.at[w], idx_v)

        bufs = (buf0, buf1)
        sems = (sem0, sem1)
        pltpu.async_copy(tab_hbm.at[idx_v.at[0]], buf0, sem0)
        pltpu.async_copy(tab_hbm.at[idx_v.at[1]], buf1, sem1)

        def step(g, carry):
            for b in range(2):
                grp = 2 * g + b
                pltpu.make_async_copy(tab_hbm.at[idx_v.at[0]], bufs[b], sems[b]).wait()
                for a in range(GRP):
                    base = a * MAX_NB

                    def rbody(r, accs, b=b, base=base):
                        out = []
                        for col in range(HIDDEN // 16):
                            acc = accs[col]
                            for rr in range(4):
                                acc = acc + bufs[b][base + r * 4 + rr,
                                                    pl.ds(col * 16, 16)]
                            out.append(acc)
                        return tuple(out)

                    accs = lax.fori_loop(
                        0, MAX_NB // 4, rbody,
                        tuple(jnp.zeros((16,), jnp.float32)
                              for _ in range(HIDDEN // 16)),
                    )
                    for col in range(HIDDEN // 16):
                        out_v[grp * GRP + a, pl.ds(col * 16, 16)] = accs[col]

                @pl.when(g < NG // 2 - 1)
                def _(b=b, grp=grp):
                    pltpu.async_copy(tab_hbm.at[idx_v.at[grp + 2]], bufs[b], sems[b])

            return carry

        lax.fori_loop(0, NG // 2, step, 0)
        pltpu.sync_copy(out_v, out_hbm.at[w])

    return gather_sum


_sc_gather_sum_h = _make_gather_sum(N_PAD)
_sc_gather_sum_fb = _make_gather_sum(TOTAL_BONDS)


_TC_ROWS = 1280
_TC_GRID = N_PAD // _TC_ROWS
_FB_ROWS = 6400
_FB_GRID = TOTAL_BONDS // _FB_ROWS


def _tc_fbproj_body(fbt_ref, wb_ref, out_ref):
    out_ref[...] = lax.dot_general(
        fbt_ref[...], wb_ref[...], (((0,), (1,)), ((), ())),
        preferred_element_type=jnp.float32)


def _tc_fbproj(fb_t, w_b):
    return pl.pallas_call(
        _tc_fbproj_body,
        grid=(_FB_GRID,),
        in_specs=[
            pl.BlockSpec((BOND_FDIM, _FB_ROWS), lambda i: (0, i)),
            pl.BlockSpec((HIDDEN, BOND_FDIM), lambda i: (0, 0)),
        ],
        out_specs=pl.BlockSpec((_FB_ROWS, HIDDEN), lambda i: (i, 0)),
        out_shape=jax.ShapeDtypeStruct((TOTAL_BONDS, HIDDEN), jnp.float32),
    )(fb_t, w_b)


def _tc_h0_body(fat_ref, wnin_ref, h0_ref):
    h0_ref[...] = jnp.maximum(
        lax.dot_general(fat_ref[...], wnin_ref[...], (((1,), (1,)), ((), ())),
                        preferred_element_type=jnp.float32),
        0.0,
    )


def _tc_h0(fatoms_pad, w_nin):
    return pl.pallas_call(
        _tc_h0_body,
        grid=(_TC_GRID,),
        in_specs=[
            pl.BlockSpec((_TC_ROWS, ATOM_FDIM), lambda i: (i, 0)),
            pl.BlockSpec((HIDDEN, ATOM_FDIM), lambda i: (0, 0)),
        ],
        out_specs=pl.BlockSpec((_TC_ROWS, HIDDEN), lambda i: (i, 0)),
        out_shape=jax.ShapeDtypeStruct((N_PAD, HIDDEN), jnp.float32),
    )(fatoms_pad, w_nin)


def _tc_update_body(nei_ref, h0_ref, bias_ref, wh_ref, out_ref):
    i = pl.program_id(0)
    y = jnp.maximum(
        h0_ref[...] + bias_ref[...]
        + lax.dot_general(nei_ref[...], wh_ref[...], (((1,), (1,)), ((), ())),
                          preferred_element_type=jnp.float32),
        0.0,
    )
    rows = lax.broadcasted_iota(jnp.int32, (_TC_ROWS, 1), 0) + i * _TC_ROWS
    out_ref[...] = jnp.where(rows < N_ATOMS, y, 0.0)


def _tc_update(nei, h0, bias, w_h):
    return pl.pallas_call(
        _tc_update_body,
        grid=(_TC_GRID,),
        in_specs=[
            pl.BlockSpec((_TC_ROWS, HIDDEN), lambda i: (i, 0)),
            pl.BlockSpec((_TC_ROWS, HIDDEN), lambda i: (i, 0)),
            pl.BlockSpec((_TC_ROWS, HIDDEN), lambda i: (i, 0)),
            pl.BlockSpec((HIDDEN, HIDDEN), lambda i: (0, 0)),
        ],
        out_specs=pl.BlockSpec((_TC_ROWS, HIDDEN), lambda i: (i, 0)),
        out_shape=jax.ShapeDtypeStruct((N_PAD, HIDDEN), jnp.float32),
    )(nei, h0, bias, w_h)


def kernel(fatoms, fbonds, aoutgraph, bgraph, aingraph, scope, all_bonds, W_nin, W_node):
    fatoms_pad = jnp.pad(fatoms, ((0, N_PAD - N_ATOMS), (0, 0)))
    aout_pad = jnp.pad(aoutgraph, ((0, N_PAD - N_ATOMS), (0, 0)))
    aout_g = aout_pad.reshape(NW, NG, IDX_W)
    spread = (jnp.arange(N_PAD * MAX_NB, dtype=jnp.int32) % TOTAL_BONDS
              ).reshape(N_PAD, MAX_NB)
    rows = jnp.arange(N_PAD, dtype=jnp.int32)[:, None]
    aout_bias = jnp.where(rows < N_ATOMS, aout_pad, spread).reshape(NW, NG, IDX_W)
    w_h = W_node[:, :HIDDEN]
    w_b = W_node[:, HIDDEN:]

    h0 = _tc_h0(fatoms_pad, W_nin)
    src = _sc_compose(aout_g, all_bonds.T)
    nei1 = _sc_gather_sum_h(h0, src)
    w_b_b, _ = lax.optimization_barrier((w_b, h0))
    fb_proj = _tc_fbproj(fbonds.T, w_b_b)
    bias = _sc_gather_sum_fb(fb_proj, aout_bias).reshape(N_PAD, HIDDEN)

    h = _tc_update(nei1.reshape(N_PAD, HIDDEN), h0, bias, w_h)
    for _ in range(DEPTH - 1):
        nei = _sc_gather_sum_h(h, src)
        h = _tc_update(nei.reshape(N_PAD, HIDDEN), h0, bias, w_h)

    return h[:N_ATOMS].T
